# Initial kernel scaffold; baseline (speedup 1.0000x reference)
import jax
import jax.numpy as jnp
from jax.experimental import pallas as pl


def kernel(tar_x, tar_loc_orig, tar_idx_agg, tar_agg_weight, src_x, src_idx_agg, src_conf, map_h, map_w, Wq, bq, Wkv, bkv, Wp, bp):
    raise NotImplementedError("write your pallas kernel here")


# speedup vs baseline: 81.6388x; 81.6388x over previous
import functools

import jax
import jax.numpy as jnp
import numpy as np
from jax.experimental import pallas as pl

B, N, C = 4, 2048, 192
N0, Ns = 4096, 2048
H, W = 64, 64
NUM_HEADS = 8
HD = C // NUM_HEADS
HWW = 7
NH = 10
PAD_OFF = 3
G = H * W
NW = NH * NH
WPAD = 128
CE = 256


def _win_of_grid():
    t = np.arange(G)
    y, x = t // W, t % W
    w = ((y + PAD_OFF) // HWW) * NH + (x + PAD_OFF) // HWW
    return w.astype(np.int32).reshape(1, 1, G)


_RBLK = 256


def _route_body(idxw_ref, agg_ref, aw_ref, out_ref):
    nb = pl.program_id(1)
    agg = agg_ref[0]
    aw = aw_ref[0]
    n_iota = jax.lax.broadcasted_iota(jnp.int32, (_RBLK, N0), 0) + nb * _RBLK
    oh_n = (agg == n_iota).astype(jnp.float32)
    iw = idxw_ref[0]
    w_iota = jax.lax.broadcasted_iota(jnp.int32, (N0, WPAD), 1)
    wv = jnp.where(iw == w_iota, aw, 0.0)
    votes = jax.lax.dot_general(oh_n, wv, (((1,), (0,)), ((), ())),
                                precision=jax.lax.Precision.HIGHEST,
                                preferred_element_type=jnp.float32)
    m = jnp.max(votes, axis=1, keepdims=True)
    cand = jnp.where(votes == m,
                     jax.lax.broadcasted_iota(jnp.int32, (_RBLK, WPAD), 1),
                     jnp.int32(2 ** 30))
    out_ref[0] = jnp.min(cand, axis=1, keepdims=True)


def _route(idx_tmp, agg, aw):
    grid = (B, N // _RBLK)
    return pl.pallas_call(
        _route_body,
        grid=grid,
        in_specs=[
            pl.BlockSpec((1, N0, 1), lambda b, n: (b, 0, 0)),
            pl.BlockSpec((1, 1, N0), lambda b, n: (b, 0, 0)),
            pl.BlockSpec((1, N0, 1), lambda b, n: (b, 0, 0)),
        ],
        out_specs=pl.BlockSpec((1, _RBLK, 1), lambda b, n: (b, n, 0)),
        out_shape=jax.ShapeDtypeStruct((B, N, 1), jnp.int32),
    )(idx_tmp, agg, aw)


_TCHUNK = 512
_NCHUNK = N0 // _TCHUNK


def _t2m_body(sidx_ref, ihw_ref, src_ref, out_ref):
    c = pl.program_id(1)

    @pl.when(c == 0)
    def _init():
        out_ref[0] = jnp.zeros((G, CE), jnp.float32)

    sidx = sidx_ref[0]
    ihw = ihw_ref[0]
    src = src_ref[0]
    s_iota = jax.lax.broadcasted_iota(jnp.int32, (_TCHUNK, Ns), 1)
    oh_s = (sidx == s_iota).astype(jnp.float32)
    gathered = jnp.dot(oh_s, src, precision=jax.lax.Precision.HIGHEST,
                       preferred_element_type=jnp.float32)
    g_iota = jax.lax.broadcasted_iota(jnp.int32, (_TCHUNK, G), 1)
    oh_g = (ihw == g_iota).astype(jnp.float32)
    acc = jax.lax.dot_general(oh_g, gathered, (((0,), (0,)), ((), ())),
                              precision=jax.lax.Precision.HIGHEST,
                              preferred_element_type=jnp.float32)
    out_ref[0] += acc

    @pl.when(c == _NCHUNK - 1)
    def _norm():
        g = out_ref[0]
        cnt = g[:, C + 1:C + 2] + 1e-6
        out_ref[0] = g / cnt


def _token2map(sidx, ihw, src_ext):
    grid = (B, _NCHUNK)
    return pl.pallas_call(
        _t2m_body,
        grid=grid,
        in_specs=[
            pl.BlockSpec((1, _TCHUNK, 1), lambda b, c: (b, c, 0)),
            pl.BlockSpec((1, _TCHUNK, 1), lambda b, c: (b, c, 0)),
            pl.BlockSpec((1, Ns, CE), lambda b, c: (b, 0, 0)),
        ],
        out_specs=pl.BlockSpec((1, G, CE), lambda b, c: (b, 0, 0)),
        out_shape=jax.ShapeDtypeStruct((B, G, CE), jnp.float32),
    )(sidx, ihw, src_ext)


def _mm_body(scale, x_ref, w_ref, b_ref, out_ref):
    x = x_ref[0]
    y = jnp.dot(x, w_ref[...], preferred_element_type=jnp.float32)
    y = y + b_ref[...]
    if scale != 1.0:
        y = y * scale
    out_ref[0] = y


def _matmul(x, w, b, mblk, scale=1.0):
    Bx, M, K = x.shape
    Nc = w.shape[1]
    grid = (Bx, M // mblk)
    return pl.pallas_call(
        functools.partial(_mm_body, scale),
        grid=grid,
        in_specs=[
            pl.BlockSpec((1, mblk, K), lambda b_, m: (b_, m, 0)),
            pl.BlockSpec((K, Nc), lambda b_, m: (0, 0)),
            pl.BlockSpec((1, Nc), lambda b_, m: (0, 0)),
        ],
        out_specs=pl.BlockSpec((1, mblk, Nc), lambda b_, m: (b_, m, 0)),
        out_shape=jax.ShapeDtypeStruct((Bx, M, Nc), jnp.float32),
    )(x, w, b)


_ABLK = 256


def _attn_body(q_ref, k_ref, v_ref, conf_ref, widx_ref, wot_ref, out_ref):
    q = q_ref[0, 0]
    k = k_ref[0, 0]
    v = v_ref[0, 0]
    conf = conf_ref[0]
    widx = widx_ref[0]
    wot = wot_ref[0]
    logits = jax.lax.dot_general(q, k, (((1,), (1,)), ((), ())),
                                 preferred_element_type=jnp.float32)
    lg = logits + conf
    lg = jnp.where(wot == widx, lg, jnp.float32(-1e30))
    m = jnp.max(lg, axis=1, keepdims=True)
    p = jnp.exp(lg - m)
    s = jnp.sum(p, axis=1, keepdims=True)
    o = jnp.dot(p, v, preferred_element_type=jnp.float32)
    out_ref[0, 0] = o / s


def _attention(q4, k4, v4, conf, widx):
    grid = (B, NUM_HEADS, N // _ABLK)
    return pl.pallas_call(
        _attn_body,
        grid=grid,
        in_specs=[
            pl.BlockSpec((1, 1, _ABLK, HD), lambda b, h, n: (b, h, n, 0)),
            pl.BlockSpec((1, 1, G, HD), lambda b, h, n: (b, h, 0, 0)),
            pl.BlockSpec((1, 1, G, HD), lambda b, h, n: (b, h, 0, 0)),
            pl.BlockSpec((1, 1, G), lambda b, h, n: (b, 0, 0)),
            pl.BlockSpec((1, _ABLK, 1), lambda b, h, n: (b, n, 0)),
            pl.BlockSpec((1, 1, G), lambda b, h, n: (0, 0, 0)),
        ],
        out_specs=pl.BlockSpec((1, 1, _ABLK, HD), lambda b, h, n: (b, h, n, 0)),
        out_shape=jax.ShapeDtypeStruct((B, NUM_HEADS, N, HD), jnp.float32),
    )(q4, k4, v4, conf, widx, jnp.asarray(_win_of_grid()))


def kernel(tar_x, tar_loc_orig, tar_idx_agg, tar_agg_weight, src_x,
           src_idx_agg, src_conf, map_h, map_w, Wq, bq, Wkv, bkv, Wp, bp):
    whf = jnp.stack([map_w, map_h]).astype(jnp.float32)

    loc = tar_loc_orig
    xy = 0.5 * (loc + 1.0) * whf[None, None, :] - 0.5
    xg = jnp.clip(jnp.round(xy[..., 0]).astype(jnp.int32), 0, W - 1)
    yg = jnp.clip(jnp.round(xy[..., 1]).astype(jnp.int32), 0, H - 1)
    idx_tmp = ((yg + PAD_OFF) // HWW) * NH + (xg + PAD_OFF) // HWW

    locc = jnp.clip(loc, -1.0, 1.0)
    locc = 0.5 * (locc + 1.0) * whf[None, None, :] - 0.5
    lx = jnp.clip(jnp.round(locc[..., 0]).astype(jnp.int32), 0, W - 1)
    ly = jnp.clip(jnp.round(locc[..., 1]).astype(jnp.int32), 0, H - 1)
    idx_hw = lx + ly * W

    widx = _route(idx_tmp.reshape(B, N0, 1),
                  tar_idx_agg.astype(jnp.int32).reshape(B, 1, N0),
                  tar_agg_weight)

    src_ext = jnp.concatenate(
        [src_x, src_conf, jnp.ones((B, Ns, 1), jnp.float32),
         jnp.zeros((B, Ns, CE - C - 2), jnp.float32)], axis=-1)
    gridm = _token2map(src_idx_agg.astype(jnp.int32).reshape(B, N0, 1),
                       idx_hw.reshape(B, N0, 1), src_ext)
    gx = gridm[..., :C]
    conf = gridm[..., C].reshape(B, 1, G)

    scale = HD ** (-0.5)
    q = _matmul(tar_x, Wq, bq.reshape(1, C), 512, scale=scale)
    kv = _matmul(gx, Wkv, bkv.reshape(1, 2 * C), 512)
    q4 = q.reshape(B, N, NUM_HEADS, HD).transpose(0, 2, 1, 3)
    k4 = kv[..., :C].reshape(B, G, NUM_HEADS, HD).transpose(0, 2, 1, 3)
    v4 = kv[..., C:].reshape(B, G, NUM_HEADS, HD).transpose(0, 2, 1, 3)

    att4 = _attention(q4, k4, v4, conf, widx)
    att = att4.transpose(0, 2, 1, 3).reshape(B, N, C)

    return _matmul(att, Wp, bp.reshape(1, C), 512)


# speedup vs baseline: 142.5572x; 1.7462x over previous
import functools

import jax
import jax.numpy as jnp
import numpy as np
from jax.experimental import pallas as pl

B, N, C = 4, 2048, 192
N0, Ns = 4096, 2048
H, W = 64, 64
NUM_HEADS = 8
HD = C // NUM_HEADS
HWW = 7
NH = 10
PAD_OFF = 3
G = H * W
NW = NH * NH
WPAD = 128
CE = 256


def _win_of_grid():
    t = np.arange(G)
    y, x = t // W, t % W
    w = ((y + PAD_OFF) // HWW) * NH + (x + PAD_OFF) // HWW
    return w.astype(np.int32).reshape(1, 1, G)


_RBLK = 256


def _route_body(idxw_ref, agg_ref, aw_ref, out_ref):
    nb = pl.program_id(1)
    agg = agg_ref[0]
    aw = aw_ref[0]
    n_iota = jax.lax.broadcasted_iota(jnp.int32, (_RBLK, N0), 0) + nb * _RBLK
    oh_n = (agg == n_iota).astype(jnp.float32)
    iw = idxw_ref[0]
    w_iota = jax.lax.broadcasted_iota(jnp.int32, (N0, WPAD), 1)
    wv = jnp.where(iw == w_iota, aw, 0.0)
    votes = jax.lax.dot_general(oh_n, wv, (((1,), (0,)), ((), ())),
                                precision=jax.lax.Precision.HIGHEST,
                                preferred_element_type=jnp.float32)
    m = jnp.max(votes, axis=1, keepdims=True)
    cand = jnp.where(votes == m,
                     jax.lax.broadcasted_iota(jnp.int32, (_RBLK, WPAD), 1),
                     jnp.int32(2 ** 30))
    out_ref[0] = jnp.min(cand, axis=1, keepdims=True)


def _route(idx_tmp, agg, aw):
    grid = (B, N // _RBLK)
    return pl.pallas_call(
        _route_body,
        grid=grid,
        in_specs=[
            pl.BlockSpec((1, N0, 1), lambda b, n: (b, 0, 0)),
            pl.BlockSpec((1, 1, N0), lambda b, n: (b, 0, 0)),
            pl.BlockSpec((1, N0, 1), lambda b, n: (b, 0, 0)),
        ],
        out_specs=pl.BlockSpec((1, _RBLK, 1), lambda b, n: (b, n, 0)),
        out_shape=jax.ShapeDtypeStruct((B, N, 1), jnp.int32),
    )(idx_tmp, agg, aw)


_TCHUNK = 512
_NCHUNK = N0 // _TCHUNK


def _t2m_body(sidx_ref, ihw_ref, src_ref, out_ref):
    c = pl.program_id(1)

    @pl.when(c == 0)
    def _init():
        out_ref[0] = jnp.zeros((G, CE), jnp.float32)

    sidx = sidx_ref[0]
    ihw = ihw_ref[0]
    src = src_ref[0]
    s_iota = jax.lax.broadcasted_iota(jnp.int32, (_TCHUNK, Ns), 1)
    oh_s = (sidx == s_iota).astype(jnp.float32)
    gathered = jnp.dot(oh_s, src, preferred_element_type=jnp.float32)
    g_iota = jax.lax.broadcasted_iota(jnp.int32, (_TCHUNK, G), 1)
    oh_g = (ihw == g_iota).astype(jnp.float32)
    acc = jax.lax.dot_general(oh_g, gathered, (((0,), (0,)), ((), ())),
                              preferred_element_type=jnp.float32)
    out_ref[0] += acc

    @pl.when(c == _NCHUNK - 1)
    def _norm():
        g = out_ref[0]
        cnt = g[:, C + 1:C + 2] + 1e-6
        out_ref[0] = g / cnt


def _token2map(sidx, ihw, src_ext):
    grid = (B, _NCHUNK)
    return pl.pallas_call(
        _t2m_body,
        grid=grid,
        in_specs=[
            pl.BlockSpec((1, _TCHUNK, 1), lambda b, c: (b, c, 0)),
            pl.BlockSpec((1, _TCHUNK, 1), lambda b, c: (b, c, 0)),
            pl.BlockSpec((1, Ns, CE), lambda b, c: (b, 0, 0)),
        ],
        out_specs=pl.BlockSpec((1, G, CE), lambda b, c: (b, 0, 0)),
        out_shape=jax.ShapeDtypeStruct((B, G, CE), jnp.float32),
    )(sidx, ihw, src_ext)


def _mm_body(scale, x_ref, w_ref, b_ref, out_ref):
    x = x_ref[0]
    y = jnp.dot(x, w_ref[...], preferred_element_type=jnp.float32)
    y = y + b_ref[...]
    if scale != 1.0:
        y = y * scale
    out_ref[0] = y


def _matmul(x, w, b, mblk, scale=1.0):
    Bx, M, K = x.shape
    Nc = w.shape[1]
    grid = (Bx, M // mblk)
    return pl.pallas_call(
        functools.partial(_mm_body, scale),
        grid=grid,
        in_specs=[
            pl.BlockSpec((1, mblk, K), lambda b_, m: (b_, m, 0)),
            pl.BlockSpec((K, Nc), lambda b_, m: (0, 0)),
            pl.BlockSpec((1, Nc), lambda b_, m: (0, 0)),
        ],
        out_specs=pl.BlockSpec((1, mblk, Nc), lambda b_, m: (b_, m, 0)),
        out_shape=jax.ShapeDtypeStruct((Bx, M, Nc), jnp.float32),
    )(x, w, b)


_ABLK = 256


def _attn_body(q_ref, k_ref, v_ref, conf_ref, widx_ref, wot_ref, out_ref):
    q = q_ref[0, 0]
    k = k_ref[0, 0]
    v = v_ref[0, 0]
    conf = conf_ref[0]
    widx = widx_ref[0]
    wot = wot_ref[0]
    logits = jax.lax.dot_general(q.astype(jnp.bfloat16),
                                 k.astype(jnp.bfloat16),
                                 (((1,), (1,)), ((), ())),
                                 preferred_element_type=jnp.float32)
    lg = jnp.where(wot == widx, logits + conf, jnp.float32(-1e30))
    p = jnp.exp(lg)
    s = jnp.sum(p, axis=1, keepdims=True)
    o = jnp.dot(p.astype(jnp.bfloat16), v.astype(jnp.bfloat16),
                preferred_element_type=jnp.float32)
    out_ref[0, 0] = o / s


def _attention(q4, k4, v4, conf, widx):
    grid = (B, NUM_HEADS, N // _ABLK)
    return pl.pallas_call(
        _attn_body,
        grid=grid,
        in_specs=[
            pl.BlockSpec((1, 1, _ABLK, HD), lambda b, h, n: (b, h, n, 0)),
            pl.BlockSpec((1, 1, G, HD), lambda b, h, n: (b, h, 0, 0)),
            pl.BlockSpec((1, 1, G, HD), lambda b, h, n: (b, h, 0, 0)),
            pl.BlockSpec((1, 1, G), lambda b, h, n: (b, 0, 0)),
            pl.BlockSpec((1, _ABLK, 1), lambda b, h, n: (b, n, 0)),
            pl.BlockSpec((1, 1, G), lambda b, h, n: (0, 0, 0)),
        ],
        out_specs=pl.BlockSpec((1, 1, _ABLK, HD), lambda b, h, n: (b, h, n, 0)),
        out_shape=jax.ShapeDtypeStruct((B, NUM_HEADS, N, HD), jnp.float32),
    )(q4, k4, v4, conf, widx, jnp.asarray(_win_of_grid()))


def kernel(tar_x, tar_loc_orig, tar_idx_agg, tar_agg_weight, src_x,
           src_idx_agg, src_conf, map_h, map_w, Wq, bq, Wkv, bkv, Wp, bp):
    whf = jnp.stack([map_w, map_h]).astype(jnp.float32)

    loc = tar_loc_orig
    xy = 0.5 * (loc + 1.0) * whf[None, None, :] - 0.5
    xg = jnp.clip(jnp.round(xy[..., 0]).astype(jnp.int32), 0, W - 1)
    yg = jnp.clip(jnp.round(xy[..., 1]).astype(jnp.int32), 0, H - 1)
    idx_tmp = ((yg + PAD_OFF) // HWW) * NH + (xg + PAD_OFF) // HWW

    locc = jnp.clip(loc, -1.0, 1.0)
    locc = 0.5 * (locc + 1.0) * whf[None, None, :] - 0.5
    lx = jnp.clip(jnp.round(locc[..., 0]).astype(jnp.int32), 0, W - 1)
    ly = jnp.clip(jnp.round(locc[..., 1]).astype(jnp.int32), 0, H - 1)
    idx_hw = lx + ly * W

    widx = _route(idx_tmp.reshape(B, N0, 1),
                  tar_idx_agg.astype(jnp.int32).reshape(B, 1, N0),
                  tar_agg_weight)

    src_ext = jnp.concatenate(
        [src_x, src_conf, jnp.ones((B, Ns, 1), jnp.float32),
         jnp.zeros((B, Ns, CE - C - 2), jnp.float32)], axis=-1)
    gridm = _token2map(src_idx_agg.astype(jnp.int32).reshape(B, N0, 1),
                       idx_hw.reshape(B, N0, 1), src_ext)
    gx = gridm[..., :C]
    conf = gridm[..., C].reshape(B, 1, G)

    scale = HD ** (-0.5)
    q = _matmul(tar_x, Wq, bq.reshape(1, C), 512, scale=scale)
    kv = _matmul(gx, Wkv, bkv.reshape(1, 2 * C), 512)
    q4 = q.reshape(B, N, NUM_HEADS, HD).transpose(0, 2, 1, 3)
    k4 = kv[..., :C].reshape(B, G, NUM_HEADS, HD).transpose(0, 2, 1, 3)
    v4 = kv[..., C:].reshape(B, G, NUM_HEADS, HD).transpose(0, 2, 1, 3)

    att4 = _attention(q4, k4, v4, conf, widx)
    att = att4.transpose(0, 2, 1, 3).reshape(B, N, C)

    return _matmul(att, Wp, bp.reshape(1, C), 512)
